# baseline (speedup 1.0000x reference)
import jax
import jax.numpy as jnp
from jax.experimental import pallas as pl
from jax.experimental.pallas import tpu as pltpu

N = 10000
D = 128
CHUNK = 200
NCHUNKS = N // CHUNK
NBUF = 4


def _body(adj_hbm, x_ref, w_ref, b_ref, out_ref, buf_ref, sems):
    def start(c):
        slot = jax.lax.rem(c, NBUF)
        pltpu.make_async_copy(
            adj_hbm.at[pl.ds(c * CHUNK, CHUNK), :],
            buf_ref.at[slot],
            sems.at[slot],
        ).start()

    for c in range(NBUF - 1):
        start(c)

    def step(c, carry):
        slot = jax.lax.rem(c, NBUF)

        @pl.when(c + NBUF - 1 < NCHUNKS)
        def _prefetch():
            start(c + NBUF - 1)

        pltpu.make_async_copy(
            adj_hbm.at[pl.ds(c * CHUNK, CHUNK), :],
            buf_ref.at[slot],
            sems.at[slot],
        ).wait()
        h = jnp.dot(
            buf_ref[slot], x_ref[...], preferred_element_type=jnp.float32
        )
        out_ref[pl.ds(c * CHUNK, CHUNK), :] = (
            jnp.dot(h, w_ref[...], preferred_element_type=jnp.float32)
            + b_ref[...]
        )
        return carry

    jax.lax.fori_loop(0, NCHUNKS, step, 0)


@jax.jit
def kernel(input, adj, A, B, weight, bias):
    bias2d = bias.reshape(1, D)
    out = pl.pallas_call(
        _body,
        in_specs=[
            pl.BlockSpec(memory_space=pltpu.MemorySpace.HBM),
            pl.BlockSpec(memory_space=pltpu.VMEM),
            pl.BlockSpec(memory_space=pltpu.VMEM),
            pl.BlockSpec(memory_space=pltpu.VMEM),
        ],
        out_specs=pl.BlockSpec(memory_space=pltpu.VMEM),
        out_shape=jax.ShapeDtypeStruct((N, D), jnp.float32),
        scratch_shapes=[
            pltpu.VMEM((NBUF, CHUNK, N), jnp.float32),
            pltpu.SemaphoreType.DMA((NBUF,)),
        ],
    )(adj, input, weight, bias2d)
    return out
